# baseline (speedup 1.0000x reference)
import functools

import jax
import jax.numpy as jnp
from jax import lax
from jax.experimental import pallas as pl
from jax.experimental.pallas import tpu as pltpu
from jax.experimental.pallas import tpu_sc as plsc

D = 64
SEQ = 200
BATCH = 4096

NC = 2
NS = 16
NW = NC * NS
BPW = BATCH // NW
DT = D // 8
NBG = 4
LOOK = 3
NBS = 2


def _body(xt_hbm, tok_hbm, pos_hbm, out_hbm, idx_v, pos_v,
          rg0, rg1, rg2, rg3, bk0, bk1, sg0, sg1, sg2, sg3, ss0, ss1):
    rows = [rg0, rg1, rg2, rg3]
    blk = [bk0, bk1]
    sem_g = [sg0, sg1, sg2, sg3]
    sem_s = [ss0, ss1]
    wid = lax.axis_index("s") * NC + lax.axis_index("c")

    pltpu.sync_copy(xt_hbm.at[pl.ds(0, SEQ), pl.ds(wid * BPW, BPW)], idx_v)
    pltpu.sync_copy(pos_hbm, pos_v)

    iota = lax.iota(jnp.int32, 16)
    tr_q = [(16 * q + iota) // 8 for q in range(4)]
    in_q = [((16 * q + iota) % 8) * 128 for q in range(4)]

    def fire_gather(s, g):
        pltpu.async_copy(tok_hbm.at[idx_v.at[s]], rows[g], sem_g[g])

    def wait_gather(g):
        pltpu.make_async_copy(tok_hbm.at[idx_v.at[0]], rows[g],
                              sem_g[g]).wait()

    def fire_store(s, k):
        pltpu.async_copy(blk[k], out_hbm.at[s, :, wid], sem_s[k])

    def wait_store(k):
        pltpu.make_async_copy(blk[k], out_hbm.at[0, :, wid], sem_s[k]).wait()

    for f in range(LOOK):
        fire_gather(f, f)

    def outer(gq, carry):
        for b in range(4):
            s = gq * 4 + b
            g = b
            k = b % NBS

            @pl.when(s + LOOK < SEQ)
            def _():
                fire_gather(s + LOOK, (b + LOOK) % NBG)

            wait_gather(g)

            @pl.when(s >= NBS)
            def _():
                wait_store(k)

            p = [pos_v[s, pl.ds(16 * q, 16)] for q in range(4)]

            @plsc.parallel_loop(0, BPW // 2, unroll=4)
            def trans(j, k=k, g=g):
                for m in range(8):
                    q, hi = m % 4, m // 4
                    c = 2 * j + hi
                    val = rows[g][c, pl.ds(16 * q, 16)] + p[q]
                    plsc.store_scatter(blk[k], [tr_q[q], in_q[q] + c], val)

            fire_store(s, k)
        return carry

    lax.fori_loop(0, SEQ // 4, outer, 0)
    for k in range(NBS):
        wait_store(k)


@jax.jit
def _emb(xt, tok, pos):
    mesh = plsc.VectorSubcoreMesh(core_axis_name="c", subcore_axis_name="s")
    kfn = functools.partial(
        pl.kernel,
        mesh=mesh,
        out_type=jax.ShapeDtypeStruct((SEQ, DT, NW, 1024), jnp.float32),
        scratch_types=(
            [pltpu.VMEM((SEQ, BPW), jnp.int32),
             pltpu.VMEM((SEQ, D), jnp.float32)]
            + [pltpu.VMEM((BPW, D), jnp.float32)] * NBG
            + [pltpu.VMEM((DT, 1024), jnp.float32)] * NBS
            + [pltpu.SemaphoreType.DMA] * (NBG + NBS)
        ),
        compiler_params=pltpu.CompilerParams(use_tc_tiling_on_sc=False,
                                             needs_layout_passes=False),
    )(_body)
    return kfn(xt, tok, pos)


def kernel(x, token_embeddings, position_embeddings):
    xt = x.astype(jnp.int32).T
    y = _emb(xt, token_embeddings, position_embeddings)
    y5 = y.reshape(SEQ, DT, NW, 8, 128)
    out = y5.transpose(2, 4, 0, 1, 3).reshape(BATCH, SEQ, D)
    return out


# speedup vs baseline: 1.7559x; 1.7559x over previous
import functools

import jax
import jax.numpy as jnp
from jax import lax
from jax.experimental import pallas as pl
from jax.experimental.pallas import tpu as pltpu
from jax.experimental.pallas import tpu_sc as plsc

D = 64
SEQ = 200
BATCH = 4096

NC = 2
NS = 16
NW = NC * NS
BPW = BATCH // NW
DT = D // 8
NBG = 4
LOOK = 3
NBS = 2


def _body(xt_hbm, tok_hbm, pos_hbm, out_hbm, idx_v, pos_v,
          rg0, rg1, rg2, rg3, bk0, bk1, sg0, sg1, sg2, sg3, ss0, ss1):
    rows = [rg0, rg1, rg2, rg3]
    blk = [bk0, bk1]
    sem_g = [sg0, sg1, sg2, sg3]
    sem_s = [ss0, ss1]
    wid = lax.axis_index("s") * NC + lax.axis_index("c")

    pltpu.sync_copy(xt_hbm.at[pl.ds(0, SEQ), pl.ds(wid * BPW, BPW)], idx_v)
    pltpu.sync_copy(pos_hbm, pos_v)

    iota = lax.iota(jnp.int32, 16)
    iota128 = iota * 128
    d_q = [16 * q + iota for q in range(4)]

    def fire_gather(s, g):
        pltpu.async_copy(tok_hbm.at[idx_v.at[s]], rows[g], sem_g[g])

    def wait_gather(g):
        pltpu.make_async_copy(tok_hbm.at[idx_v.at[0]], rows[g],
                              sem_g[g]).wait()

    def fire_store(s, k):
        for tr in range(DT):
            pltpu.async_copy(blk[k].at[pl.ds(tr * 1024, 1024)],
                             out_hbm.at[s, tr, wid], sem_s[k])

    def wait_store(k):
        for tr in range(DT):
            pltpu.make_async_copy(blk[k].at[pl.ds(tr * 1024, 1024)],
                                  out_hbm.at[0, tr, wid], sem_s[k]).wait()

    for f in range(LOOK):
        fire_gather(f, f)

    def outer(gq, carry):
        for b in range(4):
            s = gq * 4 + b
            g = b
            k = b % NBS

            @pl.when(s + LOOK < SEQ)
            def _():
                fire_gather(s + LOOK, (b + LOOK) % NBG)

            wait_gather(g)

            @pl.when(s >= NBS)
            def _():
                wait_store(k)

            p = [pos_v[s, pl.ds(16 * q, 16)] for q in range(4)]

            @plsc.parallel_loop(0, BPW, unroll=4)
            def trans(it, k=k, g=g):
                t = it & 15
                c0 = (it >> 4) * 16
                patc = (iota + t) & 15
                pats = iota128 + (patc + c0)
                cv = patc + c0
                for q in range(4):
                    val = plsc.load_gather(rows[g], [cv, d_q[q]]) + p[q]
                    plsc.store_scatter(blk[k], [pats + (2048 * q)], val)

            fire_store(s, k)
        return carry

    lax.fori_loop(0, SEQ // 4, outer, 0)
    for k in range(NBS):
        wait_store(k)


@jax.jit
def _emb(xt, tok, pos):
    mesh = plsc.VectorSubcoreMesh(core_axis_name="c", subcore_axis_name="s")
    kfn = functools.partial(
        pl.kernel,
        mesh=mesh,
        out_type=jax.ShapeDtypeStruct((SEQ, DT, NW, 1024), jnp.float32),
        scratch_types=(
            [pltpu.VMEM((SEQ, BPW), jnp.int32),
             pltpu.VMEM((SEQ, D), jnp.float32)]
            + [pltpu.VMEM((BPW, D), jnp.float32)] * NBG
            + [pltpu.VMEM((DT * 1024,), jnp.float32)] * NBS
            + [pltpu.SemaphoreType.DMA] * (NBG + NBS)
        ),
        compiler_params=pltpu.CompilerParams(use_tc_tiling_on_sc=False,
                                             needs_layout_passes=False),
    )(_body)
    return kfn(xt, tok, pos)


def kernel(x, token_embeddings, position_embeddings):
    xt = x.astype(jnp.int32).T
    y = _emb(xt, token_embeddings, position_embeddings)
    y5 = y.reshape(SEQ, DT, NW, 8, 128)
    out = y5.transpose(2, 4, 0, 1, 3).reshape(BATCH, SEQ, D)
    return out


# speedup vs baseline: 3.7797x; 2.1526x over previous
import functools

import jax
import jax.numpy as jnp
from jax import lax
from jax.experimental import pallas as pl
from jax.experimental.pallas import tpu as pltpu
from jax.experimental.pallas import tpu_sc as plsc

D = 64
SEQ = 200
BATCH = 4096
VOCAB = 1000000

NC = 2
NS = 16
NW = NC * NS
BPW = BATCH // NW
DT = D // 8
NBG = 4
LOOK = 3
NBS = 2

FCH = 256
NFCH = (VOCAB // 128) // 2
TAIL = VOCAB - NFCH * FCH
FPW_MAX = (NFCH + NW - 1) // NW


def _fmt_body(tokt_hbm, tail_hbm, s_hbm, tail_v, bi0, bi1, bi2, bo0, bo1,
              sr0, sr1, sr2, sw0, sw1):
    bin_ = [bi0, bi1, bi2]
    bout = [bo0, bo1]
    sem_r = [sr0, sr1, sr2]
    sem_w = [sw0, sw1]
    wid = lax.axis_index("s") * NC + lax.axis_index("c")

    @pl.when(wid == 0)
    def _():
        pltpu.sync_copy(tail_hbm, tail_v)
        pltpu.sync_copy(tail_v, s_hbm.at[pl.ds(NFCH * FCH * D, TAIL * D)])

    iota = lax.iota(jnp.int32, 16)
    d_q = [16 * q + iota for q in range(4)]
    n_w = (NFCH - 1 - wid) // NW + 1

    def fire_read(i, b):
        j = wid + NW * i
        pltpu.async_copy(tokt_hbm.at[pl.ds(0, D), pl.ds(FCH * j, FCH)],
                         bin_[b], sem_r[b])

    def wait_read(b):
        pltpu.make_async_copy(tokt_hbm.at[pl.ds(0, D), pl.ds(0, FCH)],
                              bin_[b], sem_r[b]).wait()

    def wait_write(b):
        pltpu.make_async_copy(bout[b], s_hbm.at[pl.ds(0, FCH * D)],
                              sem_w[b]).wait()

    fire_read(0, 0)
    fire_read(1, 1)

    def outer(gi, carry):
        for b in range(6):
            i = gi * 6 + b
            rb = b % 3
            wb = b % 2

            @pl.when(i < n_w)
            def _():
                @pl.when(i + 2 < n_w)
                def _():
                    fire_read(i + 2, (b + 2) % 3)

                wait_read(rb)

                @pl.when(i >= 2)
                def _():
                    wait_write(wb)

                @plsc.parallel_loop(0, FCH, unroll=4)
                def trans(it, rb=rb, wb=wb):
                    t = it & 15
                    c0 = (it >> 4) * 16
                    patc = (iota + t) & 15
                    cv = patc + c0
                    ps = (patc << 6) + iota + (c0 << 6)
                    for q in range(4):
                        val = plsc.load_gather(bin_[rb], [d_q[q], cv])
                        plsc.store_scatter(bout[wb], [ps + (16 * q)], val)

                j = wid + NW * i
                pltpu.async_copy(bout[wb],
                                 s_hbm.at[pl.ds(FCH * D * j, FCH * D)],
                                 sem_w[wb])
        return carry

    lax.fori_loop(0, (FPW_MAX + 5) // 6, outer, 0)

    wait_write(0)
    wait_write(1)


def _body(xt_hbm, tok_hbm, pos_hbm, out_hbm, idx_v, pos_v,
          rg0, rg1, rg2, rg3, bk0, bk1,
          sg0, sg1, sg2, sg3, ss0, ss1):
    rows = [rg0, rg1, rg2, rg3]
    blk = [bk0, bk1]
    sem_g = [sg0, sg1, sg2, sg3]
    sem_s = [ss0, ss1]
    wid = lax.axis_index("s") * NC + lax.axis_index("c")

    pltpu.sync_copy(xt_hbm.at[pl.ds(0, SEQ), pl.ds(wid * BPW, BPW)], idx_v)
    pltpu.sync_copy(pos_hbm, pos_v)

    iota = lax.iota(jnp.int32, 16)
    iota128 = iota * 128
    d_q = [16 * q + iota for q in range(4)]

    def fire_gather(s, g):
        pltpu.async_copy(tok_hbm.at[idx_v.at[s]], rows[g], sem_g[g])

    def wait_gather(g):
        pltpu.make_async_copy(tok_hbm.at[idx_v.at[0]], rows[g],
                              sem_g[g]).wait()

    def fire_store(s, k):
        for tr in range(DT):
            pltpu.async_copy(blk[k].at[pl.ds(tr * 1024, 1024)],
                             out_hbm.at[s, tr, wid], sem_s[k])

    def wait_store(k):
        for tr in range(DT):
            pltpu.make_async_copy(blk[k].at[pl.ds(tr * 1024, 1024)],
                                  out_hbm.at[0, tr, wid], sem_s[k]).wait()

    for f in range(LOOK):
        fire_gather(f, f)

    def outer(gq, carry):
        for b in range(NBG):
            s = gq * NBG + b
            g = b
            k = b % NBS

            @pl.when(s < SEQ)
            def _():
                @pl.when(s + LOOK < SEQ)
                def _():
                    fire_gather(s + LOOK, (b + LOOK) % NBG)

                wait_gather(g)

                @pl.when(s >= NBS)
                def _():
                    wait_store(k)

                p = [pos_v[s, pl.ds(16 * q, 16)] for q in range(4)]

                @plsc.parallel_loop(0, BPW, unroll=4)
                def trans(it, k=k, g=g):
                    t = it & 15
                    c0 = (it >> 4) * 16
                    patc = (iota + t) & 15
                    pats = iota128 + (patc + c0)
                    cv = patc + c0
                    for q in range(4):
                        val = plsc.load_gather(rows[g], [cv, d_q[q]]) + p[q]
                        plsc.store_scatter(blk[k], [pats + (2048 * q)], val)

                fire_store(s, k)
        return carry

    lax.fori_loop(0, (SEQ + NBG - 1) // NBG, outer, 0)
    for k in range(NBS):
        wait_store(k)


@jax.jit
def _emb(xt, tokt, tail, pos):
    mesh = plsc.VectorSubcoreMesh(core_axis_name="c", subcore_axis_name="s")
    fmt = functools.partial(
        pl.kernel,
        mesh=mesh,
        out_type=jax.ShapeDtypeStruct((VOCAB * D,), jnp.float32),
        scratch_types=(
            [pltpu.VMEM((TAIL * D,), jnp.float32)]
            + [pltpu.VMEM((D, FCH), jnp.float32)] * 3
            + [pltpu.VMEM((FCH * D,), jnp.float32)] * 2
            + [pltpu.SemaphoreType.DMA] * 5
        ),
        compiler_params=pltpu.CompilerParams(use_tc_tiling_on_sc=True,
                                             needs_layout_passes=False),
    )(_fmt_body)
    s1d = fmt(tokt, tail)
    tok = s1d.reshape(VOCAB, D)
    kfn = functools.partial(
        pl.kernel,
        mesh=mesh,
        out_type=jax.ShapeDtypeStruct((SEQ, DT, NW, 1024), jnp.float32),
        scratch_types=(
            [pltpu.VMEM((SEQ, BPW), jnp.int32),
             pltpu.VMEM((SEQ, D), jnp.float32)]
            + [pltpu.VMEM((BPW, D), jnp.float32)] * NBG
            + [pltpu.VMEM((DT * 1024,), jnp.float32)] * NBS
            + [pltpu.SemaphoreType.DMA] * (NBG + NBS)
        ),
        compiler_params=pltpu.CompilerParams(use_tc_tiling_on_sc=False,
                                             needs_layout_passes=False),
    )(_body)
    return kfn(xt, tok, pos)


def kernel(x, token_embeddings, position_embeddings):
    xt = x.astype(jnp.int32).T
    tokt = token_embeddings.T
    tail = token_embeddings[NFCH * FCH:, :].reshape(TAIL * D)
    y = _emb(xt, tokt, tail, position_embeddings)
    y5 = y.reshape(SEQ, DT, NW, 8, 128)
    out = y5.transpose(2, 4, 0, 1, 3).reshape(BATCH, SEQ, D)
    return out


# speedup vs baseline: 3.8593x; 1.0210x over previous
import functools

import jax
import jax.numpy as jnp
from jax import lax
from jax.experimental import pallas as pl
from jax.experimental.pallas import tpu as pltpu
from jax.experimental.pallas import tpu_sc as plsc

D = 64
SEQ = 200
BATCH = 4096
VOCAB = 1000000

NC = 2
NS = 16
NW = NC * NS
BPW = BATCH // NW
DT = D // 8
NBG = 4
LOOK = 3
NBS = 2

FCH = 384
NFCH = (VOCAB // 128) // 3
TAIL = VOCAB - NFCH * FCH
FPW_MAX = (NFCH + NW - 1) // NW


def _fmt_body(tokt_hbm, tail_hbm, s_hbm, tail_v, bi0, bi1, bo0, bo1,
              sr0, sr1, sw0, sw1):
    bin_ = [bi0, bi1]
    bout = [bo0, bo1]
    sem_r = [sr0, sr1]
    sem_w = [sw0, sw1]
    wid = lax.axis_index("s") * NC + lax.axis_index("c")

    @pl.when(wid == 0)
    def _():
        pltpu.sync_copy(tail_hbm, tail_v)
        pltpu.sync_copy(tail_v, s_hbm.at[pl.ds(NFCH * FCH * D, TAIL * D)])

    iota = lax.iota(jnp.int32, 16)
    d_q = [16 * q + iota for q in range(4)]
    n_w = (NFCH - 1 - wid) // NW + 1

    def fire_read(i, b):
        j = wid + NW * i
        pltpu.async_copy(tokt_hbm.at[pl.ds(0, D), pl.ds(FCH * j, FCH)],
                         bin_[b], sem_r[b])

    def wait_read(b):
        pltpu.make_async_copy(tokt_hbm.at[pl.ds(0, D), pl.ds(0, FCH)],
                              bin_[b], sem_r[b]).wait()

    def wait_write(b):
        pltpu.make_async_copy(bout[b], s_hbm.at[pl.ds(0, FCH * D)],
                              sem_w[b]).wait()

    fire_read(0, 0)

    def outer(gi, carry):
        for b in range(2):
            i = gi * 2 + b
            rb = b % 2
            wb = b % 2

            @pl.when(i < n_w)
            def _():
                @pl.when(i + 1 < n_w)
                def _():
                    fire_read(i + 1, (b + 1) % 2)

                wait_read(rb)

                @pl.when(i >= 2)
                def _():
                    wait_write(wb)

                @plsc.parallel_loop(0, FCH, unroll=4)
                def trans(it, rb=rb, wb=wb):
                    t = it & 15
                    c0 = (it >> 4) * 16
                    patc = (iota + t) & 15
                    cv = patc + c0
                    ps = (patc << 6) + iota + (c0 << 6)
                    for q in range(4):
                        val = plsc.load_gather(bin_[rb], [d_q[q], cv])
                        plsc.store_scatter(bout[wb], [ps + (16 * q)], val)

                j = wid + NW * i
                pltpu.async_copy(bout[wb],
                                 s_hbm.at[pl.ds(FCH * D * j, FCH * D)],
                                 sem_w[wb])
        return carry

    lax.fori_loop(0, (FPW_MAX + 1) // 2, outer, 0)

    wait_write(0)
    wait_write(1)


def _body(xt_hbm, tok_hbm, pos_hbm, out_hbm, idx_v, pos_v,
          rg0, rg1, rg2, rg3, bk0, bk1,
          sg0, sg1, sg2, sg3, ss0, ss1):
    rows = [rg0, rg1, rg2, rg3]
    blk = [bk0, bk1]
    sem_g = [sg0, sg1, sg2, sg3]
    sem_s = [ss0, ss1]
    wid = lax.axis_index("s") * NC + lax.axis_index("c")

    pltpu.sync_copy(xt_hbm.at[pl.ds(0, SEQ), pl.ds(wid * BPW, BPW)], idx_v)
    pltpu.sync_copy(pos_hbm, pos_v)

    iota = lax.iota(jnp.int32, 16)
    iota128 = iota * 128
    d_q = [16 * q + iota for q in range(4)]

    def fire_gather(s, g):
        pltpu.async_copy(tok_hbm.at[idx_v.at[s]], rows[g], sem_g[g])

    def wait_gather(g):
        pltpu.make_async_copy(tok_hbm.at[idx_v.at[0]], rows[g],
                              sem_g[g]).wait()

    def fire_store(s, k):
        for tr in range(DT):
            pltpu.async_copy(blk[k].at[pl.ds(tr * 1024, 1024)],
                             out_hbm.at[s, tr, wid], sem_s[k])

    def wait_store(k):
        for tr in range(DT):
            pltpu.make_async_copy(blk[k].at[pl.ds(tr * 1024, 1024)],
                                  out_hbm.at[0, tr, wid], sem_s[k]).wait()

    for f in range(LOOK):
        fire_gather(f, f)

    def outer(gq, carry):
        for b in range(NBG):
            s = gq * NBG + b
            g = b
            k = b % NBS

            @pl.when(s < SEQ)
            def _():
                @pl.when(s + LOOK < SEQ)
                def _():
                    fire_gather(s + LOOK, (b + LOOK) % NBG)

                wait_gather(g)

                @pl.when(s >= NBS)
                def _():
                    wait_store(k)

                p = [pos_v[s, pl.ds(16 * q, 16)] for q in range(4)]

                @plsc.parallel_loop(0, BPW, unroll=4)
                def trans(it, k=k, g=g):
                    t = it & 15
                    c0 = (it >> 4) * 16
                    patc = (iota + t) & 15
                    pats = iota128 + (patc + c0)
                    cv = patc + c0
                    for q in range(4):
                        val = plsc.load_gather(rows[g], [cv, d_q[q]]) + p[q]
                        plsc.store_scatter(blk[k], [pats + (2048 * q)], val)

                fire_store(s, k)
        return carry

    lax.fori_loop(0, (SEQ + NBG - 1) // NBG, outer, 0)
    for k in range(NBS):
        wait_store(k)


@jax.jit
def _emb(xt, tokt, tail, pos):
    mesh = plsc.VectorSubcoreMesh(core_axis_name="c", subcore_axis_name="s")
    fmt = functools.partial(
        pl.kernel,
        mesh=mesh,
        out_type=jax.ShapeDtypeStruct((VOCAB * D,), jnp.float32),
        scratch_types=(
            [pltpu.VMEM((TAIL * D,), jnp.float32)]
            + [pltpu.VMEM((D, FCH), jnp.float32)] * 2
            + [pltpu.VMEM((FCH * D,), jnp.float32)] * 2
            + [pltpu.SemaphoreType.DMA] * 4
        ),
        compiler_params=pltpu.CompilerParams(use_tc_tiling_on_sc=True,
                                             needs_layout_passes=False),
    )(_fmt_body)
    s1d = fmt(tokt, tail)
    tok = s1d.reshape(VOCAB, D)
    kfn = functools.partial(
        pl.kernel,
        mesh=mesh,
        out_type=jax.ShapeDtypeStruct((SEQ, DT, NW, 1024), jnp.float32),
        scratch_types=(
            [pltpu.VMEM((SEQ, BPW), jnp.int32),
             pltpu.VMEM((SEQ, D), jnp.float32)]
            + [pltpu.VMEM((BPW, D), jnp.float32)] * NBG
            + [pltpu.VMEM((DT * 1024,), jnp.float32)] * NBS
            + [pltpu.SemaphoreType.DMA] * (NBG + NBS)
        ),
        compiler_params=pltpu.CompilerParams(use_tc_tiling_on_sc=False,
                                             needs_layout_passes=False),
    )(_body)
    return kfn(xt, tok, pos)


def kernel(x, token_embeddings, position_embeddings):
    xt = x.astype(jnp.int32).T
    tokt = token_embeddings.T
    tail = token_embeddings[NFCH * FCH:, :].reshape(TAIL * D)
    y = _emb(xt, tokt, tail, position_embeddings)
    y5 = y.reshape(SEQ, DT, NW, 8, 128)
    out = y5.transpose(2, 4, 0, 1, 3).reshape(BATCH, SEQ, D)
    return out
